# baseline (speedup 1.0000x reference)
import functools

import jax
import jax.numpy as jnp
from jax import lax
from jax.experimental import pallas as pl
from jax.experimental.pallas import tpu as pltpu
from jax.experimental.pallas import tpu_sc as plsc

_NC = 2
_NS = 16
_NW = _NC * _NS
_L = 16

_D = 1024
_R = 32768
_CH = 16
_RPW = _R // _NW
_NCHUNK = _RPW // _CH
_NBUF = 4


def _sc_add_row(x_hbm, tbl_hbm, idx_hbm, out_hbm,
                idx_v, row_v, b0, b1, b2, b3, *sems):
    bufs = (b0, b1, b2, b3)
    gsem = sems[0]
    isems = sems[1:5]
    osems = sems[5:9]

    wid = lax.axis_index("s") * _NC + lax.axis_index("c")
    base = wid * _RPW

    pltpu.sync_copy(idx_hbm, idx_v)
    pltpu.async_copy(tbl_hbm.at[idx_v], row_v, gsem).wait()

    def start_in(c, b):
        pltpu.async_copy(x_hbm.at[pl.ds(base + c * _CH, _CH)], bufs[b], isems[b])

    def wait_in(c, b):
        pltpu.make_async_copy(
            x_hbm.at[pl.ds(base + c * _CH, _CH)], bufs[b], isems[b]).wait()

    def start_out(c, b):
        pltpu.async_copy(bufs[b], out_hbm.at[pl.ds(base + c * _CH, _CH)], osems[b])

    def wait_out(c, b):
        pltpu.make_async_copy(
            bufs[b], out_hbm.at[pl.ds(base + c * _CH, _CH)], osems[b]).wait()

    def compute(b):
        buf = bufs[b]

        def jbody(j, _):
            col = j * _L
            rv = row_v[0, pl.ds(col, _L)]
            for i in range(_CH):
                buf[i, pl.ds(col, _L)] = buf[i, pl.ds(col, _L)] + rv
            return 0

        lax.fori_loop(0, _D // _L, jbody, 0, unroll=4)

    for b in range(_NBUF):
        start_in(b, b)

    def gbody(g, _):
        c0 = _NBUF * g
        for b in range(_NBUF):
            wait_in(c0 + b, b)
            compute(b)
            start_out(c0 + b, b)
        for b in range(_NBUF):
            wait_out(c0 + b, b)
            start_in(c0 + b + _NBUF, b)
        return 0

    lax.fori_loop(0, _NCHUNK // _NBUF - 1, gbody, 0)

    cl = _NCHUNK - _NBUF
    for b in range(_NBUF):
        wait_in(cl + b, b)
        compute(b)
        start_out(cl + b, b)
    for b in range(_NBUF):
        wait_out(cl + b, b)


def kernel(x, tf_table, tf_id):
    B, S, D = x.shape
    R = B * S
    xf = x.reshape(R, D)
    idx = jnp.full((_L,), tf_id, dtype=jnp.int32)

    mesh = plsc.VectorSubcoreMesh(core_axis_name="c", subcore_axis_name="s")
    run = functools.partial(
        pl.kernel,
        mesh=mesh,
        out_type=jax.ShapeDtypeStruct((R, D), x.dtype),
        scratch_types=[
            pltpu.VMEM((_L,), jnp.int32),
            pltpu.VMEM((_L, _D), jnp.float32),
            pltpu.VMEM((_CH, _D), jnp.float32),
            pltpu.VMEM((_CH, _D), jnp.float32),
            pltpu.VMEM((_CH, _D), jnp.float32),
            pltpu.VMEM((_CH, _D), jnp.float32),
        ] + [pltpu.SemaphoreType.DMA] * 9,
    )(_sc_add_row)
    out = run(xf, tf_table, idx)
    return out.reshape(B, S, D)


# speedup vs baseline: 2.6283x; 2.6283x over previous
import functools

import jax
import jax.numpy as jnp
from jax import lax
from jax.experimental import pallas as pl
from jax.experimental.pallas import tpu as pltpu
from jax.experimental.pallas import tpu_sc as plsc

_NC = 2
_NS = 16
_NW = _NC * _NS
_L = 16

_D = 1024
_R = 32768
_CH = 16
_RPW = _R // _NW
_NCHUNK = _RPW // _CH


def _sc_add_row(x_hbm, tbl_hbm, idx_hbm, out_hbm,
                idx_v, row_v, in0, in1, ot0, ot1,
                gsem, isem0, isem1, osem0, osem1):
    wid = lax.axis_index("s") * _NC + lax.axis_index("c")
    base = wid * _RPW

    pltpu.sync_copy(idx_hbm, idx_v)
    pltpu.async_copy(tbl_hbm.at[idx_v], row_v, gsem).wait()

    def start_in(c, buf, sem):
        pltpu.async_copy(x_hbm.at[pl.ds(base + c * _CH, _CH)], buf, sem)

    def wait_in(c, buf, sem):
        pltpu.make_async_copy(x_hbm.at[pl.ds(base + c * _CH, _CH)], buf, sem).wait()

    def start_out(c, buf, sem):
        pltpu.async_copy(buf, out_hbm.at[pl.ds(base + c * _CH, _CH)], sem)

    def wait_out(c, buf, sem):
        pltpu.make_async_copy(buf, out_hbm.at[pl.ds(base + c * _CH, _CH)], sem).wait()

    def compute(inb, outb):
        @plsc.parallel_loop(0, _D // _L, unroll=4)
        def jbody(j):
            col = j * _L
            rv = row_v[0, pl.ds(col, _L)]
            for i in range(_CH):
                outb[i, pl.ds(col, _L)] = inb[i, pl.ds(col, _L)] + rv

    start_in(0, in0, isem0)
    start_in(1, in1, isem1)

    wait_in(0, in0, isem0)
    compute(in0, ot0)
    start_in(2, in0, isem0)
    start_out(0, ot0, osem0)

    wait_in(1, in1, isem1)
    compute(in1, ot1)
    start_in(3, in1, isem1)
    start_out(1, ot1, osem1)

    def gbody(g, _):
        c0 = 2 * g
        wait_out(c0 - 2, ot0, osem0)
        wait_in(c0, in0, isem0)
        compute(in0, ot0)
        start_in(c0 + 2, in0, isem0)
        start_out(c0, ot0, osem0)

        c1 = c0 + 1
        wait_out(c1 - 2, ot1, osem1)
        wait_in(c1, in1, isem1)
        compute(in1, ot1)
        start_in(c1 + 2, in1, isem1)
        start_out(c1, ot1, osem1)
        return 0

    lax.fori_loop(1, _NCHUNK // 2 - 1, gbody, 0)

    cl0 = _NCHUNK - 2
    wait_out(cl0 - 2, ot0, osem0)
    wait_in(cl0, in0, isem0)
    compute(in0, ot0)
    start_out(cl0, ot0, osem0)

    cl1 = _NCHUNK - 1
    wait_out(cl1 - 2, ot1, osem1)
    wait_in(cl1, in1, isem1)
    compute(in1, ot1)
    start_out(cl1, ot1, osem1)

    wait_out(cl0, ot0, osem0)
    wait_out(cl1, ot1, osem1)


def kernel(x, tf_table, tf_id):
    B, S, D = x.shape
    R = B * S
    xf = x.reshape(R, D)
    idx = jnp.full((_L,), tf_id, dtype=jnp.int32)

    mesh = plsc.VectorSubcoreMesh(core_axis_name="c", subcore_axis_name="s")
    run = functools.partial(
        pl.kernel,
        mesh=mesh,
        out_type=jax.ShapeDtypeStruct((R, D), x.dtype),
        scratch_types=[
            pltpu.VMEM((_L,), jnp.int32),
            pltpu.VMEM((_L, _D), jnp.float32),
            pltpu.VMEM((_CH, _D), jnp.float32),
            pltpu.VMEM((_CH, _D), jnp.float32),
            pltpu.VMEM((_CH, _D), jnp.float32),
            pltpu.VMEM((_CH, _D), jnp.float32),
            pltpu.SemaphoreType.DMA,
            pltpu.SemaphoreType.DMA,
            pltpu.SemaphoreType.DMA,
            pltpu.SemaphoreType.DMA,
            pltpu.SemaphoreType.DMA,
        ],
    )(_sc_add_row)
    out = run(xf, tf_table, idx)
    return out.reshape(B, S, D)


# speedup vs baseline: 2.6632x; 1.0133x over previous
import functools

import jax
import jax.numpy as jnp
from jax import lax
from jax.experimental import pallas as pl
from jax.experimental.pallas import tpu as pltpu
from jax.experimental.pallas import tpu_sc as plsc

_NC = 2
_NS = 16
_NW = _NC * _NS
_L = 16

_D = 1024
_R = 32768
_CH = 16
_RPW = _R // _NW
_NCHUNK = _RPW // _CH


def _sc_add_row(x_hbm, tbl_hbm, idx_hbm, out_hbm,
                idx_v, row_v, in0, in1, ot0, ot1,
                gsem, isem0, isem1, osem0, osem1):
    wid = lax.axis_index("s") * _NC + lax.axis_index("c")
    base = wid * _RPW

    pltpu.sync_copy(idx_hbm, idx_v)
    pltpu.async_copy(tbl_hbm.at[idx_v], row_v, gsem).wait()

    def start_in(c, buf, sem):
        pltpu.async_copy(x_hbm.at[pl.ds(base + c * _CH, _CH)], buf, sem)

    def wait_in(c, buf, sem):
        pltpu.make_async_copy(x_hbm.at[pl.ds(base + c * _CH, _CH)], buf, sem).wait()

    def start_out(c, buf, sem):
        pltpu.async_copy(buf, out_hbm.at[pl.ds(base + c * _CH, _CH)], sem)

    def wait_out(c, buf, sem):
        pltpu.make_async_copy(buf, out_hbm.at[pl.ds(base + c * _CH, _CH)], sem).wait()

    def compute(inb, outb):
        @plsc.parallel_loop(0, _D // _L, unroll=8)
        def jbody(j):
            col = j * _L
            rv = row_v[0, pl.ds(col, _L)]
            for i in range(_CH):
                outb[i, pl.ds(col, _L)] = inb[i, pl.ds(col, _L)] + rv

    start_in(0, in0, isem0)
    start_in(1, in1, isem1)

    wait_in(0, in0, isem0)
    compute(in0, ot0)
    start_in(2, in0, isem0)
    start_out(0, ot0, osem0)

    wait_in(1, in1, isem1)
    compute(in1, ot1)
    start_in(3, in1, isem1)
    start_out(1, ot1, osem1)

    def gbody(g, _):
        c0 = 2 * g
        wait_out(c0 - 2, ot0, osem0)
        wait_in(c0, in0, isem0)
        compute(in0, ot0)
        start_in(c0 + 2, in0, isem0)
        start_out(c0, ot0, osem0)

        c1 = c0 + 1
        wait_out(c1 - 2, ot1, osem1)
        wait_in(c1, in1, isem1)
        compute(in1, ot1)
        start_in(c1 + 2, in1, isem1)
        start_out(c1, ot1, osem1)
        return 0

    lax.fori_loop(1, _NCHUNK // 2 - 1, gbody, 0)

    cl0 = _NCHUNK - 2
    wait_out(cl0 - 2, ot0, osem0)
    wait_in(cl0, in0, isem0)
    compute(in0, ot0)
    start_out(cl0, ot0, osem0)

    cl1 = _NCHUNK - 1
    wait_out(cl1 - 2, ot1, osem1)
    wait_in(cl1, in1, isem1)
    compute(in1, ot1)
    start_out(cl1, ot1, osem1)

    wait_out(cl0, ot0, osem0)
    wait_out(cl1, ot1, osem1)


def kernel(x, tf_table, tf_id):
    B, S, D = x.shape
    R = B * S
    xf = x.reshape(R, D)
    idx = jnp.full((_L,), tf_id, dtype=jnp.int32)

    mesh = plsc.VectorSubcoreMesh(core_axis_name="c", subcore_axis_name="s")
    run = functools.partial(
        pl.kernel,
        mesh=mesh,
        out_type=jax.ShapeDtypeStruct((R, D), x.dtype),
        scratch_types=[
            pltpu.VMEM((_L,), jnp.int32),
            pltpu.VMEM((_L, _D), jnp.float32),
            pltpu.VMEM((_CH, _D), jnp.float32),
            pltpu.VMEM((_CH, _D), jnp.float32),
            pltpu.VMEM((_CH, _D), jnp.float32),
            pltpu.VMEM((_CH, _D), jnp.float32),
            pltpu.SemaphoreType.DMA,
            pltpu.SemaphoreType.DMA,
            pltpu.SemaphoreType.DMA,
            pltpu.SemaphoreType.DMA,
            pltpu.SemaphoreType.DMA,
        ],
    )(_sc_add_row)
    out = run(xf, tf_table, idx)
    return out.reshape(B, S, D)


# speedup vs baseline: 3.5490x; 1.3326x over previous
import functools

import jax
import jax.numpy as jnp
from jax import lax
from jax.experimental import pallas as pl
from jax.experimental.pallas import tpu as pltpu
from jax.experimental.pallas import tpu_sc as plsc

_NC = 2
_D = 1024


def _sc_lookup(tbl_hbm, idx_hbm, out_hbm, idx_v, row_v, sem):
    wid = lax.axis_index("s") * _NC + lax.axis_index("c")

    @pl.when(wid == 0)
    def _():
        pltpu.sync_copy(idx_hbm, idx_v)
        pltpu.async_copy(tbl_hbm.at[idx_v], row_v, sem).wait()
        pltpu.sync_copy(row_v, out_hbm)


def _tc_add(row_ref, x_ref, o_ref):
    o_ref[...] = x_ref[...] + row_ref[0]


def kernel(x, tf_table, tf_id):
    B, S, D = x.shape
    R = B * S
    xf = x.reshape(R, D)
    idx = jnp.full((1,), tf_id, dtype=jnp.int32)

    mesh = plsc.VectorSubcoreMesh(core_axis_name="c", subcore_axis_name="s")
    row = functools.partial(
        pl.kernel,
        mesh=mesh,
        out_type=jax.ShapeDtypeStruct((1, _D), jnp.float32),
        scratch_types=[
            pltpu.VMEM((1,), jnp.int32),
            pltpu.VMEM((1, _D), jnp.float32),
            pltpu.SemaphoreType.DMA,
        ],
    )(_sc_lookup)(tf_table, idx)

    BLK = 2048
    out = pl.pallas_call(
        _tc_add,
        grid=(R // BLK,),
        in_specs=[
            pl.BlockSpec((1, D), lambda i: (0, 0)),
            pl.BlockSpec((BLK, D), lambda i: (i, 0)),
        ],
        out_specs=pl.BlockSpec((BLK, D), lambda i: (i, 0)),
        out_shape=jax.ShapeDtypeStruct((R, D), x.dtype),
        compiler_params=pltpu.CompilerParams(
            dimension_semantics=("arbitrary",),
        ),
    )(row, xf)
    return out.reshape(B, S, D)


# speedup vs baseline: 4.2863x; 1.2077x over previous
import jax
import jax.numpy as jnp
from jax.experimental import pallas as pl
from jax.experimental.pallas import tpu as pltpu


def _add_row_body(tf_id_ref, table_ref, x_ref, o_ref):
    del tf_id_ref
    o_ref[...] = x_ref[...] + table_ref[0]


def kernel(x, tf_table, tf_id):
    B, S, D = x.shape
    R = B * S
    xf = x.reshape(R, D)
    tbl3 = tf_table.reshape(tf_table.shape[0], 1, D)
    tf_id_arr = jnp.asarray(tf_id, dtype=jnp.int32).reshape(1)

    BLK = 2048
    grid = (R // BLK,)
    out = pl.pallas_call(
        _add_row_body,
        grid_spec=pltpu.PrefetchScalarGridSpec(
            num_scalar_prefetch=1,
            grid=grid,
            in_specs=[
                pl.BlockSpec((1, 1, D), lambda i, tf_id_ref: (tf_id_ref[0], 0, 0)),
                pl.BlockSpec((BLK, D), lambda i, tf_id_ref: (i, 0)),
            ],
            out_specs=pl.BlockSpec((BLK, D), lambda i, tf_id_ref: (i, 0)),
        ),
        out_shape=jax.ShapeDtypeStruct((R, D), x.dtype),
        compiler_params=pltpu.CompilerParams(
            dimension_semantics=("parallel",),
        ),
    )(tf_id_arr, tbl3, xf)
    return out.reshape(B, S, D)


# speedup vs baseline: 4.3005x; 1.0033x over previous
import jax
import jax.numpy as jnp
from jax.experimental import pallas as pl
from jax.experimental.pallas import tpu as pltpu


def _add_row_body(tf_id_ref, table_ref, x_ref, o_ref):
    del tf_id_ref
    o_ref[...] = x_ref[...] + table_ref[0]


def kernel(x, tf_table, tf_id):
    B, S, D = x.shape
    R = B * S
    xf = x.reshape(R, D)
    tbl3 = tf_table.reshape(tf_table.shape[0], 1, D)
    tf_id_arr = jnp.asarray(tf_id, dtype=jnp.int32).reshape(1)

    BLK = 2048
    grid = (R // BLK,)
    out = pl.pallas_call(
        _add_row_body,
        grid_spec=pltpu.PrefetchScalarGridSpec(
            num_scalar_prefetch=1,
            grid=grid,
            in_specs=[
                pl.BlockSpec((1, 1, D), lambda i, tf_id_ref: (tf_id_ref[0], 0, 0)),
                pl.BlockSpec((BLK, D), lambda i, tf_id_ref: (i, 0)),
            ],
            out_specs=pl.BlockSpec((BLK, D), lambda i, tf_id_ref: (i, 0)),
        ),
        out_shape=jax.ShapeDtypeStruct((R, D), x.dtype),
        compiler_params=pltpu.CompilerParams(
            dimension_semantics=("arbitrary",),
        ),
    )(tf_id_arr, tbl3, xf)
    return out.reshape(B, S, D)
